# baseline (speedup 1.0000x reference)
import jax
import jax.numpy as jnp
from jax.experimental import pallas as pl


def _body(x_ref, p_ref, emb_ref, out_ref):
    blk, batch, d = x_ref.shape
    s = jax.lax.dot_general(
        p_ref[...], emb_ref[...],
        dimension_numbers=(((0,), (0,)), ((), ())),
        preferred_element_type=jnp.float32,
    )
    out_ref[...] = x_ref[...] + s.reshape(blk, batch, d)


def kernel(x, phase_one_hot, emb_table):
    seq, batch, d = x.shape
    n = emb_table.shape[0]
    pt = jnp.transpose(phase_one_hot, (2, 0, 1)).reshape(n, seq * batch)
    blk = 1024
    grid = (seq // blk,)
    return pl.pallas_call(
        _body,
        grid=grid,
        in_specs=[
            pl.BlockSpec((blk, batch, d), lambda i: (i, 0, 0)),
            pl.BlockSpec((n, blk * batch), lambda i: (0, i)),
            pl.BlockSpec((n, d), lambda i: (0, 0)),
        ],
        out_specs=pl.BlockSpec((blk, batch, d), lambda i: (i, 0, 0)),
        out_shape=jax.ShapeDtypeStruct((seq, batch, d), x.dtype),
    )(x, pt, emb_table)


# speedup vs baseline: 1.2408x; 1.2408x over previous
import jax
import jax.numpy as jnp
from jax.experimental import pallas as pl


def _body(x_ref, q_ref, emb_ref, out_ref):
    blk, batch, d = x_ref.shape
    n = q_ref.shape[0]
    pb = jnp.transpose(q_ref[...], (0, 2, 1)).reshape(n, blk * batch)
    s = jax.lax.dot_general(
        pb, emb_ref[...],
        dimension_numbers=(((0,), (0,)), ((), ())),
        preferred_element_type=jnp.float32,
    )
    out_ref[...] = x_ref[...] + s.reshape(blk, batch, d)


def kernel(x, phase_one_hot, emb_table):
    seq, batch, d = x.shape
    n = emb_table.shape[0]
    q = jnp.transpose(phase_one_hot, (2, 1, 0))
    blk = 512
    grid = (seq // blk,)
    return pl.pallas_call(
        _body,
        grid=grid,
        in_specs=[
            pl.BlockSpec((blk, batch, d), lambda i: (i, 0, 0)),
            pl.BlockSpec((n, batch, blk), lambda i: (0, 0, i)),
            pl.BlockSpec((n, d), lambda i: (0, 0)),
        ],
        out_specs=pl.BlockSpec((blk, batch, d), lambda i: (i, 0, 0)),
        out_shape=jax.ShapeDtypeStruct((seq, batch, d), x.dtype),
    )(x, q, emb_table)


# speedup vs baseline: 1.2542x; 1.0108x over previous
import jax
import jax.numpy as jnp
from jax.experimental import pallas as pl


def _body(x_ref, q_ref, emb_ref, out_ref):
    blk, batch, d = x_ref.shape
    n = q_ref.shape[0]
    pb = jnp.transpose(q_ref[...], (0, 2, 1)).reshape(n, blk * batch)
    s = jax.lax.dot_general(
        pb, emb_ref[...],
        dimension_numbers=(((0,), (0,)), ((), ())),
        preferred_element_type=jnp.float32,
    )
    out_ref[...] = x_ref[...] + s.reshape(blk, batch, d)


def kernel(x, phase_one_hot, emb_table):
    seq, batch, d = x.shape
    n = emb_table.shape[0]
    q = jnp.transpose(phase_one_hot, (2, 1, 0))
    blk = 1024
    grid = (seq // blk,)
    return pl.pallas_call(
        _body,
        grid=grid,
        in_specs=[
            pl.BlockSpec((blk, batch, d), lambda i: (i, 0, 0)),
            pl.BlockSpec((n, batch, blk), lambda i: (0, 0, i)),
            pl.BlockSpec((n, d), lambda i: (0, 0)),
        ],
        out_specs=pl.BlockSpec((blk, batch, d), lambda i: (i, 0, 0)),
        out_shape=jax.ShapeDtypeStruct((seq, batch, d), x.dtype),
    )(x, q, emb_table)
